# baseline (speedup 1.0000x reference)
import functools

import jax
import jax.numpy as jnp
import numpy as np
from jax import lax
from jax.experimental import pallas as pl
from jax.experimental.pallas import tpu as pltpu
from jax.experimental.pallas import tpu_sc as plsc

_B = 256
_F = 65536
_C = 4096
_NC = 2
_NS = 16
_NW = _NC * _NS
_L = 16
_PAIRS_PER_W = _B // 2 // _NW
_DUMMY = 2 * _C
_ACC = 2 * _C + _L
_SCALE = float(np.sqrt(1.0 / 0.001) / np.sqrt(_C))
_CH = 16384
_U = 16
_HI_MASK = -65536


@functools.lru_cache(maxsize=None)
def _make_sc_kernel(nchunk: int):
    mesh = plsc.VectorSubcoreMesh(core_axis_name="c", subcore_axis_name="s")

    @functools.partial(
        pl.kernel,
        mesh=mesh,
        compiler_params=pltpu.CompilerParams(needs_layout_passes=False),
        out_type=jax.ShapeDtypeStruct((_B, _C), jnp.float32),
        scratch_types=[
            pltpu.VMEM((2, _CH), jnp.int32),
            pltpu.VMEM((_F,), jnp.int32),
            pltpu.VMEM((_ACC,), jnp.float32),
            pltpu.VMEM((_ACC,), jnp.float32),
            pltpu.VMEM((_C,), jnp.float32),
            pltpu.SemaphoreType.DMA,
            pltpu.SemaphoreType.DMA,
        ],
    )
    def sc_kernel(x2_hbm, pk_hbm, out_hbm, pk_v, x2row_v, acc0_v, acc1_v,
                  outs_v, sem0, sem1):
        cid = lax.axis_index("c")
        sid = lax.axis_index("s")
        wid = sid * _NC + cid
        sems = (sem0, sem1)

        def pair_body(i, carry):
            pr = wid * _PAIRS_PER_W + i
            pltpu.sync_copy(x2_hbm.at[pl.ds(pr * _F, _F)], x2row_v)

            @plsc.parallel_loop(0, _ACC // _L, unroll=4)
            def zero_body(jj):
                z = jnp.zeros((_L,), jnp.float32)
                acc0_v[pl.ds(jj * _L, _L)] = z
                acc1_v[pl.ds(jj * _L, _L)] = z

            def chunk_work(buf_slot):
                @plsc.parallel_loop(0, _CH // _L, unroll=_U)
                def acc_body(j):
                    p = pk_v[buf_slot, pl.ds(j * _L, _L)]
                    colv = p & 0xFFFF
                    rowv = p >> 16
                    g2 = plsc.load_gather(x2row_v, [colv])
                    f_even = plsc.bitcast(g2 & _HI_MASK, jnp.float32)
                    f_odd = plsc.bitcast(g2 << 16, jnp.float32)
                    plsc.addupdate_scatter(acc0_v, [rowv], f_even)
                    plsc.addupdate_scatter(acc1_v, [rowv], f_odd)

            copies = [None, None]
            copies[0] = pltpu.async_copy(
                pk_hbm.at[pl.ds(0, _CH)], pk_v.at[0], sems[0])
            for t in range(nchunk):
                nxt = t + 1
                if nxt < nchunk:
                    copies[nxt % 2] = pltpu.async_copy(
                        pk_hbm.at[pl.ds(nxt * _CH, _CH)], pk_v.at[nxt % 2],
                        sems[nxt % 2])
                copies[t % 2].wait()
                chunk_work(t % 2)

            for half, acc_v in ((0, acc0_v), (1, acc1_v)):
                @plsc.parallel_loop(0, _C // _L, unroll=4)
                def comb_body(j, acc_v=acc_v):
                    pos = acc_v[pl.ds(j * _L, _L)]
                    neg = acc_v[pl.ds(_C + j * _L, _L)]
                    outs_v[pl.ds(j * _L, _L)] = (pos - neg) * _SCALE

                pltpu.sync_copy(outs_v, out_hbm.at[pr + half * (_B // 2)])
            return carry

        lax.fori_loop(0, _PAIRS_PER_W, pair_body, 0)

    return sc_kernel


def kernel(X, row, col, vals):
    nnz = row.shape[0]
    nchunk = -(-nnz // _CH)
    row_aug = row + _C * (vals < 0).astype(jnp.int32)
    packed = col | (row_aug << 16)
    pad = jnp.full((nchunk * _CH - nnz,), _DUMMY << 16, dtype=jnp.int32)
    packed = jnp.concatenate([packed, pad])
    xu = lax.bitcast_convert_type(X.astype(jnp.bfloat16), jnp.uint16)
    x2 = (xu[: _B // 2].astype(jnp.uint32) << 16) | xu[_B // 2:].astype(jnp.uint32)
    x2 = lax.bitcast_convert_type(x2, jnp.int32).reshape(-1)
    return _make_sc_kernel(nchunk)(x2, packed)


# speedup vs baseline: 1.1684x; 1.1684x over previous
import functools

import jax
import jax.numpy as jnp
import numpy as np
from jax import lax
from jax.experimental import pallas as pl
from jax.experimental.pallas import tpu as pltpu
from jax.experimental.pallas import tpu_sc as plsc

_B = 256
_F = 65536
_C = 4096
_NC = 2
_NS = 16
_NW = _NC * _NS
_L = 16
_PAIRS_PER_W = _B // 2 // _NW
_DUMMY = 2 * _C
_ACC = 2 * _C + _L
_SCALE = float(np.sqrt(1.0 / 0.001) / np.sqrt(_C))
_CH = 8192
_U = 4
_HI_MASK = -65536


@functools.lru_cache(maxsize=None)
def _make_sc_kernel(nchunk: int):
    mesh = plsc.VectorSubcoreMesh(core_axis_name="c", subcore_axis_name="s")

    @functools.partial(
        pl.kernel,
        mesh=mesh,
        compiler_params=pltpu.CompilerParams(needs_layout_passes=False),
        out_type=jax.ShapeDtypeStruct((_B, _C), jnp.float32),
        scratch_types=[
            pltpu.VMEM((2, _CH), jnp.int32),
            pltpu.VMEM((_F,), jnp.int32),
            pltpu.VMEM((_ACC,), jnp.float32),
            pltpu.VMEM((_ACC,), jnp.float32),
            pltpu.VMEM((_C,), jnp.float32),
            pltpu.SemaphoreType.DMA,
            pltpu.SemaphoreType.DMA,
        ],
    )
    def sc_kernel(x2_hbm, pk_hbm, out_hbm, pk_v, x2row_v, acc0_v, acc1_v,
                  outs_v, sem0, sem1):
        cid = lax.axis_index("c")
        sid = lax.axis_index("s")
        wid = sid * _NC + cid
        sems = (sem0, sem1)

        def pair_body(i, carry):
            pr = wid * _PAIRS_PER_W + i
            pltpu.sync_copy(x2_hbm.at[pl.ds(pr * _F, _F)], x2row_v)

            @plsc.parallel_loop(0, _ACC // _L, unroll=4)
            def zero_body(jj):
                z = jnp.zeros((_L,), jnp.float32)
                acc0_v[pl.ds(jj * _L, _L)] = z
                acc1_v[pl.ds(jj * _L, _L)] = z

            def chunk_work(buf_slot):
                @plsc.parallel_loop(0, _CH // _L, unroll=_U)
                def acc_body(j):
                    p = pk_v[buf_slot, pl.ds(j * _L, _L)]
                    colv = p & 0xFFFF
                    rowv = p >> 16
                    g2 = plsc.load_gather(x2row_v, [colv])
                    f_even = plsc.bitcast(g2 & _HI_MASK, jnp.float32)
                    f_odd = plsc.bitcast(g2 << 16, jnp.float32)
                    plsc.addupdate_scatter(acc0_v, [rowv], f_even)
                    plsc.addupdate_scatter(acc1_v, [rowv], f_odd)

            copies = [None, None]
            copies[0] = pltpu.async_copy(
                pk_hbm.at[pl.ds(0, _CH)], pk_v.at[0], sems[0])
            for t in range(nchunk):
                nxt = t + 1
                if nxt < nchunk:
                    copies[nxt % 2] = pltpu.async_copy(
                        pk_hbm.at[pl.ds(nxt * _CH, _CH)], pk_v.at[nxt % 2],
                        sems[nxt % 2])
                copies[t % 2].wait()
                chunk_work(t % 2)

            for half, acc_v in ((0, acc0_v), (1, acc1_v)):
                @plsc.parallel_loop(0, _C // _L, unroll=4)
                def comb_body(j, acc_v=acc_v):
                    pos = acc_v[pl.ds(j * _L, _L)]
                    neg = acc_v[pl.ds(_C + j * _L, _L)]
                    outs_v[pl.ds(j * _L, _L)] = (pos - neg) * _SCALE

                pltpu.sync_copy(outs_v, out_hbm.at[pr + half * (_B // 2)])
            return carry

        lax.fori_loop(0, _PAIRS_PER_W, pair_body, 0)

    return sc_kernel


def kernel(X, row, col, vals):
    nnz = row.shape[0]
    nchunk = -(-nnz // _CH)
    row_aug = row + _C * (vals < 0).astype(jnp.int32)
    packed = col | (row_aug << 16)
    pad = jnp.full((nchunk * _CH - nnz,), _DUMMY << 16, dtype=jnp.int32)
    packed = jnp.concatenate([packed, pad])
    xu = lax.bitcast_convert_type(X.astype(jnp.bfloat16), jnp.uint16)
    x2 = (xu[: _B // 2].astype(jnp.uint32) << 16) | xu[_B // 2:].astype(jnp.uint32)
    x2 = lax.bitcast_convert_type(x2, jnp.int32).reshape(-1)
    return _make_sc_kernel(nchunk)(x2, packed)
